# Initial kernel scaffold; baseline (speedup 1.0000x reference)
import jax
import jax.numpy as jnp
from jax.experimental import pallas as pl


def kernel(x, play_emb, hand_emb, W1, b1, W2, b2):
    raise NotImplementedError("write your pallas kernel here")


# speedup vs baseline: 59.4002x; 59.4002x over previous
import functools

import jax
import jax.numpy as jnp
from jax import lax
from jax.experimental import pallas as pl
from jax.experimental.pallas import tpu as pltpu
from jax.experimental.pallas import tpu_sc as plsc

_NPLAY = 56
_NHAND = 54
_NPOS = 3 * _NPLAY + _NHAND
_BINS = 224
_L = 16
_NW = 32


def _sc_hist_body(xr_hbm, out_hbm, xbuf, hist):
  cid = lax.axis_index("c")
  sid = lax.axis_index("s")
  wid = sid * 2 + cid
  ngroups = xr_hbm.shape[0] // (_NPOS * _L)
  groups_per_tile = ngroups // _NW

  laneoff = lax.iota(jnp.int32, _L) * _BINS
  ones = jnp.full((_L,), 1.0, dtype=jnp.float32)
  zeros = jnp.zeros((_L,), dtype=jnp.float32)

  def group_body(g, carry):
    grp = wid * groups_per_tile + g
    pltpu.sync_copy(xr_hbm.at[pl.ds(grp * _NPOS * _L, _NPOS * _L)], xbuf)
    for k in range(0, _L * _BINS, _L):
      hist[pl.ds(k, _L)] = zeros
    for j in range(_NPOS):
      off = 56 * min(j // _NPLAY, 3)
      flat = xbuf[pl.ds(j * _L, _L)] + (laneoff + off)
      plsc.addupdate_scatter(hist, [flat], ones)
    pltpu.sync_copy(hist, out_hbm.at[pl.ds(grp * _L * _BINS, _L * _BINS)])
    return carry

  lax.fori_loop(0, groups_per_tile, group_body, 0)


def _make_sc_hist(batch):
  mesh = plsc.VectorSubcoreMesh(core_axis_name="c", subcore_axis_name="s")
  return pl.kernel(
      _sc_hist_body,
      mesh=mesh,
      out_type=jax.ShapeDtypeStruct((batch * _BINS,), jnp.float32),
      scratch_types=[
          pltpu.VMEM((_NPOS * _L,), jnp.int32),
          pltpu.VMEM((_L * _BINS,), jnp.float32),
      ],
      compiler_params=pltpu.CompilerParams(needs_layout_passes=False),
  )


def _mlp_body(c_ref, xs_ref, pe_ref, he_ref, w1_ref, b1_ref, w2_ref, b2_ref,
              o_ref):
  f32 = jnp.float32
  pe = pe_ref[...]
  he = he_ref[...]
  w1 = w1_ref[...]
  dn = (((1,), (1,)), ((), ()))
  a1 = lax.dot_general(pe, w1[:, 0:12], dn, preferred_element_type=f32)
  a2 = lax.dot_general(pe, w1[:, 12:24], dn, preferred_element_type=f32)
  a3 = lax.dot_general(pe, w1[:, 24:36], dn, preferred_element_type=f32)
  a4 = lax.dot_general(he, w1[:, 36:56], dn, preferred_element_type=f32)
  pad = jnp.zeros((2, 128), dtype=f32)
  a = jnp.concatenate([a1, a2, a3, a4, pad], axis=0)

  c = c_ref[...]
  h = lax.dot_general(c, a, (((1,), (0,)), ((), ())),
                      preferred_element_type=f32)
  h = jnp.maximum(h + b1_ref[...], 0.0)
  out = lax.dot_general(h, w2_ref[...], dn, preferred_element_type=f32)
  out = out + b2_ref[...]
  skip = xs_ref[...].astype(f32)
  skip = jnp.pad(skip, ((0, 0), (0, 1)))
  o_ref[...] = out + skip


def kernel(x, play_emb, hand_emb, W1, b1, W2, b2):
  x = x.astype(jnp.int32)
  batch = x.shape[0]

  xr = x.reshape(batch // _L, _L, _NPOS).swapaxes(1, 2).reshape(-1)
  counts = _make_sc_hist(batch)(xr).reshape(batch, _BINS)

  xs = x[:, 168:222]

  bblk = 2048
  grid = (batch // bblk,)
  out = pl.pallas_call(
      _mlp_body,
      grid=grid,
      in_specs=[
          pl.BlockSpec((bblk, _BINS), lambda i: (i, 0)),
          pl.BlockSpec((bblk, _NHAND), lambda i: (i, 0)),
          pl.BlockSpec((56, 12), lambda i: (0, 0)),
          pl.BlockSpec((54, 20), lambda i: (0, 0)),
          pl.BlockSpec((128, 56), lambda i: (0, 0)),
          pl.BlockSpec((1, 128), lambda i: (0, 0)),
          pl.BlockSpec((55, 128), lambda i: (0, 0)),
          pl.BlockSpec((1, 55), lambda i: (0, 0)),
      ],
      out_specs=pl.BlockSpec((bblk, 55), lambda i: (i, 0)),
      out_shape=jax.ShapeDtypeStruct((batch, 55), jnp.float32),
  )(counts, xs, play_emb, hand_emb, W1, b1.reshape(1, 128), W2,
    b2.reshape(1, 55))
  return out


# speedup vs baseline: 73.8143x; 1.2427x over previous
import functools

import jax
import jax.numpy as jnp
from jax import lax
from jax.experimental import pallas as pl
from jax.experimental.pallas import tpu as pltpu
from jax.experimental.pallas import tpu_sc as plsc

_NPLAY = 56
_NHAND = 54
_NPOS = 3 * _NPLAY + _NHAND
_BINS = 224
_L = 16
_NW = 32


_CS = 32
_CIN = _CS * _NPOS
_COUT = _CS * _BINS


def _sc_hist_body(x_hbm, out_hbm, xb0, xb1, h0, h1, isem0, isem1, osem0,
                  osem1):
  cid = lax.axis_index("c")
  sid = lax.axis_index("s")
  wid = sid * 2 + cid
  nchunks = x_hbm.shape[0] // (_CIN * _NW)
  base = wid * nchunks

  lane = lax.iota(jnp.int32, _L)
  gath = lane * _NPOS
  scat = lane * _BINS
  ones = jnp.full((_L,), 1.0, dtype=jnp.float32)
  zeros = jnp.zeros((_L,), dtype=jnp.float32)

  def in_copy(i, xb, isem):
    return pltpu.make_async_copy(
        x_hbm.at[pl.ds((base + i) * _CIN, _CIN)], xb, isem)

  def out_copy(i, hh, osem):
    return pltpu.make_async_copy(
        hh, out_hbm.at[pl.ds((base + i) * _COUT, _COUT)], osem)

  bufs = ((xb0, h0, isem0, osem0), (xb1, h1, isem1, osem1))
  in_copy(0, xb0, isem0).start()

  def pair_body(p, carry):
    for b, (xb, hh, isem, osem) in enumerate(bufs):
      i = p * 2 + b
      nxt = bufs[1 - b]
      if b == 0:
        in_copy(i + 1, nxt[0], nxt[2]).start()
      else:
        @pl.when(p < (nchunks // 2) - 1)
        def _():
          in_copy(i + 1, nxt[0], nxt[2]).start()
      @pl.when(p >= 1)
      def _():
        out_copy(i, hh, osem).wait()
      for k in range(0, _COUT, _L):
        hh[pl.ds(k, _L)] = zeros
      in_copy(i, xb, isem).wait()
      for j in range(_NPOS):
        off = 56 * min(j // _NPLAY, 3)
        for sub in range(_CS // _L):
          xv = plsc.load_gather(xb, [gath + (sub * _L * _NPOS + j)])
          flat = xv + (scat + (sub * _L * _BINS + off))
          plsc.addupdate_scatter(hh, [flat], ones)
      out_copy(i, hh, osem).start()
    return carry

  lax.fori_loop(0, nchunks // 2, pair_body, 0)
  out_copy(nchunks - 2, h0, osem0).wait()
  out_copy(nchunks - 1, h1, osem1).wait()


def _make_sc_hist(batch):
  mesh = plsc.VectorSubcoreMesh(core_axis_name="c", subcore_axis_name="s")
  return pl.kernel(
      _sc_hist_body,
      mesh=mesh,
      out_type=jax.ShapeDtypeStruct((batch * _BINS,), jnp.float32),
      scratch_types=[
          pltpu.VMEM((_CIN,), jnp.int32),
          pltpu.VMEM((_CIN,), jnp.int32),
          pltpu.VMEM((_COUT,), jnp.float32),
          pltpu.VMEM((_COUT,), jnp.float32),
          pltpu.SemaphoreType.DMA,
          pltpu.SemaphoreType.DMA,
          pltpu.SemaphoreType.DMA,
          pltpu.SemaphoreType.DMA,
      ],
      compiler_params=pltpu.CompilerParams(needs_layout_passes=False),
  )


def _mlp_body(c_ref, xs_ref, pe_ref, he_ref, w1_ref, b1_ref, w2_ref, b2_ref,
              o_ref):
  f32 = jnp.float32
  pe = pe_ref[...]
  he = he_ref[...]
  w1 = w1_ref[...]
  dn = (((1,), (1,)), ((), ()))
  a1 = lax.dot_general(pe, w1[:, 0:12], dn, preferred_element_type=f32)
  a2 = lax.dot_general(pe, w1[:, 12:24], dn, preferred_element_type=f32)
  a3 = lax.dot_general(pe, w1[:, 24:36], dn, preferred_element_type=f32)
  a4 = lax.dot_general(he, w1[:, 36:56], dn, preferred_element_type=f32)
  pad = jnp.zeros((2, 128), dtype=f32)
  a = jnp.concatenate([a1, a2, a3, a4, pad], axis=0)

  c = c_ref[...]
  h = lax.dot_general(c, a, (((1,), (0,)), ((), ())),
                      preferred_element_type=f32)
  h = jnp.maximum(h + b1_ref[...], 0.0)
  out = lax.dot_general(h, w2_ref[...], dn, preferred_element_type=f32)
  out = out + b2_ref[...]
  skip = xs_ref[...].astype(f32)
  skip = jnp.pad(skip, ((0, 0), (0, 1)))
  o_ref[...] = out + skip


def kernel(x, play_emb, hand_emb, W1, b1, W2, b2):
  x = x.astype(jnp.int32)
  batch = x.shape[0]

  counts = _make_sc_hist(batch)(x.reshape(-1)).reshape(batch, _BINS)

  xs = x[:, 168:222]

  bblk = 2048
  grid = (batch // bblk,)
  out = pl.pallas_call(
      _mlp_body,
      grid=grid,
      in_specs=[
          pl.BlockSpec((bblk, _BINS), lambda i: (i, 0)),
          pl.BlockSpec((bblk, _NHAND), lambda i: (i, 0)),
          pl.BlockSpec((56, 12), lambda i: (0, 0)),
          pl.BlockSpec((54, 20), lambda i: (0, 0)),
          pl.BlockSpec((128, 56), lambda i: (0, 0)),
          pl.BlockSpec((1, 128), lambda i: (0, 0)),
          pl.BlockSpec((55, 128), lambda i: (0, 0)),
          pl.BlockSpec((1, 55), lambda i: (0, 0)),
      ],
      out_specs=pl.BlockSpec((bblk, 55), lambda i: (i, 0)),
      out_shape=jax.ShapeDtypeStruct((batch, 55), jnp.float32),
  )(counts, xs, play_emb, hand_emb, W1, b1.reshape(1, 128), W2,
    b2.reshape(1, 55))
  return out


# speedup vs baseline: 113.2372x; 1.5341x over previous
import functools

import jax
import jax.numpy as jnp
from jax import lax
from jax.experimental import pallas as pl
from jax.experimental.pallas import tpu as pltpu
from jax.experimental.pallas import tpu_sc as plsc

_NPLAY = 56
_NHAND = 54
_NPOS = 3 * _NPLAY + _NHAND
_BINS = 224
_L = 16
_NW = 32


_CS = 32
_CIN = _CS * _NPOS
_COUT = _CS * _BINS


def _sc_hist_body(x_hbm, out_hbm, xb0, xb1, h0, h1, isem0, isem1, osem0,
                  osem1):
  cid = lax.axis_index("c")
  sid = lax.axis_index("s")
  wid = sid * 2 + cid
  nchunks = x_hbm.shape[0] // (_CIN * _NW)
  base = wid * nchunks

  lane = lax.iota(jnp.int32, _L)
  gath = lane * _NPOS
  scat = lane * _BINS
  ones = jnp.full((_L,), 1.0, dtype=jnp.float32)
  zeros = jnp.zeros((_L,), dtype=jnp.float32)

  def in_copy(i, xb, isem):
    return pltpu.make_async_copy(
        x_hbm.at[pl.ds((base + i) * _CIN, _CIN)], xb, isem)

  def out_copy(i, hh, osem):
    return pltpu.make_async_copy(
        hh, out_hbm.at[pl.ds((base + i) * _COUT, _COUT)], osem)

  bufs = ((xb0, h0, isem0, osem0), (xb1, h1, isem1, osem1))
  in_copy(0, xb0, isem0).start()

  def pair_body(p, carry):
    for b, (xb, hh, isem, osem) in enumerate(bufs):
      i = p * 2 + b
      nxt = bufs[1 - b]
      if b == 0:
        in_copy(i + 1, nxt[0], nxt[2]).start()
      else:
        @pl.when(p < (nchunks // 2) - 1)
        def _():
          in_copy(i + 1, nxt[0], nxt[2]).start()
      @pl.when(p >= 1)
      def _():
        out_copy(i, hh, osem).wait()
      for k in range(0, _COUT, _L):
        hh[pl.ds(k, _L)] = zeros
      in_copy(i, xb, isem).wait()
      for lo, hi, unroll in ((0, 56, 8), (56, 112, 8), (112, 168, 8),
                             (168, 222, 6)):
        def pos_body(j, lo=lo):
          for sub in range(_CS // _L):
            xv = plsc.load_gather(xb, [gath + (sub * _L * _NPOS) + j])
            flat = xv + (scat + (sub * _L * _BINS + lo))
            plsc.addupdate_scatter(hh, [flat], ones)
        plsc.parallel_loop(lo, hi, unroll=unroll)(pos_body)
      out_copy(i, hh, osem).start()
    return carry

  lax.fori_loop(0, nchunks // 2, pair_body, 0)
  out_copy(nchunks - 2, h0, osem0).wait()
  out_copy(nchunks - 1, h1, osem1).wait()


def _make_sc_hist(batch):
  mesh = plsc.VectorSubcoreMesh(core_axis_name="c", subcore_axis_name="s")
  return pl.kernel(
      _sc_hist_body,
      mesh=mesh,
      out_type=jax.ShapeDtypeStruct((batch * _BINS,), jnp.float32),
      scratch_types=[
          pltpu.VMEM((_CIN,), jnp.int32),
          pltpu.VMEM((_CIN,), jnp.int32),
          pltpu.VMEM((_COUT,), jnp.float32),
          pltpu.VMEM((_COUT,), jnp.float32),
          pltpu.SemaphoreType.DMA,
          pltpu.SemaphoreType.DMA,
          pltpu.SemaphoreType.DMA,
          pltpu.SemaphoreType.DMA,
      ],
      compiler_params=pltpu.CompilerParams(needs_layout_passes=False),
  )


def _mlp_body(c_ref, xs_ref, pe_ref, he_ref, w1_ref, b1_ref, w2_ref, b2_ref,
              o_ref):
  f32 = jnp.float32
  pe = pe_ref[...]
  he = he_ref[...]
  w1 = w1_ref[...]
  dn = (((1,), (1,)), ((), ()))
  a1 = lax.dot_general(pe, w1[:, 0:12], dn, preferred_element_type=f32)
  a2 = lax.dot_general(pe, w1[:, 12:24], dn, preferred_element_type=f32)
  a3 = lax.dot_general(pe, w1[:, 24:36], dn, preferred_element_type=f32)
  a4 = lax.dot_general(he, w1[:, 36:56], dn, preferred_element_type=f32)
  pad = jnp.zeros((2, 128), dtype=f32)
  a = jnp.concatenate([a1, a2, a3, a4, pad], axis=0)

  c = c_ref[...]
  h = lax.dot_general(c, a, (((1,), (0,)), ((), ())),
                      preferred_element_type=f32)
  h = jnp.maximum(h + b1_ref[...], 0.0)
  out = lax.dot_general(h, w2_ref[...], dn, preferred_element_type=f32)
  out = out + b2_ref[...]
  skip = xs_ref[...].astype(f32)
  skip = jnp.pad(skip, ((0, 0), (0, 1)))
  o_ref[...] = out + skip


def kernel(x, play_emb, hand_emb, W1, b1, W2, b2):
  x = x.astype(jnp.int32)
  batch = x.shape[0]

  counts = _make_sc_hist(batch)(x.reshape(-1)).reshape(batch, _BINS)

  xs = x[:, 168:222]

  bblk = 2048
  grid = (batch // bblk,)
  out = pl.pallas_call(
      _mlp_body,
      grid=grid,
      in_specs=[
          pl.BlockSpec((bblk, _BINS), lambda i: (i, 0)),
          pl.BlockSpec((bblk, _NHAND), lambda i: (i, 0)),
          pl.BlockSpec((56, 12), lambda i: (0, 0)),
          pl.BlockSpec((54, 20), lambda i: (0, 0)),
          pl.BlockSpec((128, 56), lambda i: (0, 0)),
          pl.BlockSpec((1, 128), lambda i: (0, 0)),
          pl.BlockSpec((55, 128), lambda i: (0, 0)),
          pl.BlockSpec((1, 55), lambda i: (0, 0)),
      ],
      out_specs=pl.BlockSpec((bblk, 55), lambda i: (i, 0)),
      out_shape=jax.ShapeDtypeStruct((batch, 55), jnp.float32),
  )(counts, xs, play_emb, hand_emb, W1, b1.reshape(1, 128), W2,
    b2.reshape(1, 55))
  return out


# speedup vs baseline: 117.8953x; 1.0411x over previous
import functools

import jax
import jax.numpy as jnp
from jax import lax
from jax.experimental import pallas as pl
from jax.experimental.pallas import tpu as pltpu
from jax.experimental.pallas import tpu_sc as plsc

_NPLAY = 56
_NHAND = 54
_NPOS = 3 * _NPLAY + _NHAND
_BINS = 224
_L = 16
_NW = 32


_CS = 32
_CIN = _CS * _NPOS
_COUT = _CS * _BINS


def _sc_hist_body(x_hbm, out_hbm, xb0, xb1, h0, h1, isem0, isem1, osem0,
                  osem1):
  cid = lax.axis_index("c")
  sid = lax.axis_index("s")
  wid = sid * 2 + cid
  nchunks = x_hbm.shape[0] // (_CS * _NW)
  base = wid * nchunks

  lane = lax.iota(jnp.int32, _L)
  ones = jnp.full((_L,), 1.0, dtype=jnp.float32)
  zeros = jnp.zeros((_L,), dtype=jnp.float32)

  def in_copy(i, xb, isem):
    return pltpu.make_async_copy(
        x_hbm.at[pl.ds((base + i) * _CS, _CS)], xb, isem)

  def out_copy(i, hh, osem):
    return pltpu.make_async_copy(
        hh, out_hbm.at[pl.ds((base + i) * _CS, _CS)], osem)

  bufs = ((xb0, h0, isem0, osem0), (xb1, h1, isem1, osem1))
  in_copy(0, xb0, isem0).start()

  def pair_body(p, carry):
    for b, (xb, hh, isem, osem) in enumerate(bufs):
      i = p * 2 + b
      nxt = bufs[1 - b]
      if b == 0:
        in_copy(i + 1, nxt[0], nxt[2]).start()
      else:
        @pl.when(p < (nchunks // 2) - 1)
        def _():
          in_copy(i + 1, nxt[0], nxt[2]).start()
      @pl.when(p >= 1)
      def _():
        out_copy(i, hh, osem).wait()
      for r in range(_CS):
        for c0 in range(0, _BINS, _L):
          hh[r, pl.ds(c0, _L)] = zeros
      in_copy(i, xb, isem).wait()
      for lo, hi, unroll in ((0, 56, 8), (56, 112, 8), (112, 168, 8),
                             (168, 222, 6)):
        def pos_body(j, lo=lo):
          col = jnp.full((_L,), 0, jnp.int32) + j
          for sub in range(_CS // _L):
            rows = lane + sub * _L
            xv = plsc.load_gather(xb, [rows, col])
            plsc.addupdate_scatter(hh, [rows, xv + lo], ones)
        plsc.parallel_loop(lo, hi, unroll=unroll)(pos_body)
      out_copy(i, hh, osem).start()
    return carry

  lax.fori_loop(0, nchunks // 2, pair_body, 0)
  out_copy(nchunks - 2, h0, osem0).wait()
  out_copy(nchunks - 1, h1, osem1).wait()


def _make_sc_hist(batch):
  mesh = plsc.VectorSubcoreMesh(core_axis_name="c", subcore_axis_name="s")
  return pl.kernel(
      _sc_hist_body,
      mesh=mesh,
      out_type=jax.ShapeDtypeStruct((batch, _BINS), jnp.float32),
      scratch_types=[
          pltpu.VMEM((_CS, _NPOS), jnp.int32),
          pltpu.VMEM((_CS, _NPOS), jnp.int32),
          pltpu.VMEM((_CS, _BINS), jnp.float32),
          pltpu.VMEM((_CS, _BINS), jnp.float32),
          pltpu.SemaphoreType.DMA,
          pltpu.SemaphoreType.DMA,
          pltpu.SemaphoreType.DMA,
          pltpu.SemaphoreType.DMA,
      ],
      compiler_params=pltpu.CompilerParams(needs_layout_passes=False),
  )


def _mlp_body(c_ref, xs_ref, pe_ref, he_ref, w1_ref, b1_ref, w2_ref, b2_ref,
              o_ref):
  f32 = jnp.float32
  pe = pe_ref[...]
  he = he_ref[...]
  w1 = w1_ref[...]
  dn = (((1,), (1,)), ((), ()))
  a1 = lax.dot_general(pe, w1[:, 0:12], dn, preferred_element_type=f32)
  a2 = lax.dot_general(pe, w1[:, 12:24], dn, preferred_element_type=f32)
  a3 = lax.dot_general(pe, w1[:, 24:36], dn, preferred_element_type=f32)
  a4 = lax.dot_general(he, w1[:, 36:56], dn, preferred_element_type=f32)
  pad = jnp.zeros((2, 128), dtype=f32)
  a = jnp.concatenate([a1, a2, a3, a4, pad], axis=0)

  c = c_ref[...]
  h = lax.dot_general(c, a, (((1,), (0,)), ((), ())),
                      preferred_element_type=f32)
  h = jnp.maximum(h + b1_ref[...], 0.0)
  out = lax.dot_general(h, w2_ref[...], dn, preferred_element_type=f32)
  out = out + b2_ref[...]
  skip = xs_ref[:, 168:222].astype(f32)
  skip = jnp.pad(skip, ((0, 0), (0, 1)))
  o_ref[...] = out + skip


def kernel(x, play_emb, hand_emb, W1, b1, W2, b2):
  x = x.astype(jnp.int32)
  batch = x.shape[0]

  counts = _make_sc_hist(batch)(x)

  bblk = 2048
  grid = (batch // bblk,)
  out = pl.pallas_call(
      _mlp_body,
      grid=grid,
      in_specs=[
          pl.BlockSpec((bblk, _BINS), lambda i: (i, 0)),
          pl.BlockSpec((bblk, _NPOS), lambda i: (i, 0)),
          pl.BlockSpec((56, 12), lambda i: (0, 0)),
          pl.BlockSpec((54, 20), lambda i: (0, 0)),
          pl.BlockSpec((128, 56), lambda i: (0, 0)),
          pl.BlockSpec((1, 128), lambda i: (0, 0)),
          pl.BlockSpec((55, 128), lambda i: (0, 0)),
          pl.BlockSpec((1, 55), lambda i: (0, 0)),
      ],
      out_specs=pl.BlockSpec((bblk, 55), lambda i: (i, 0)),
      out_shape=jax.ShapeDtypeStruct((batch, 55), jnp.float32),
  )(counts, x, play_emb, hand_emb, W1, b1.reshape(1, 128), W2,
    b2.reshape(1, 55))
  return out
